# baseline (speedup 1.0000x reference)
import jax
import jax.numpy as jnp
from jax.experimental import pallas as pl
from jax.experimental.pallas import tpu as pltpu

B, N, D, E, K = 2, 2048, 768, 8, 2
TB = 1024
NT = B * N // TB


def _moe_block(x_ref, gw_ref, w1_ref, b1_ref, w2_ref, b2_ref, o_ref, wt_ref):
    e = pl.program_id(1)
    xb = x_ref[...]

    @pl.when(e == 0)
    def _router():
        logits = jnp.dot(xb, gw_ref[...], preferred_element_type=jnp.float32)
        probs = jax.nn.softmax(logits, axis=-1)
        e_ids = jax.lax.broadcasted_iota(jnp.int32, probs.shape, 1)
        i1 = jnp.argmax(probs, axis=-1)
        p1 = jnp.max(probs, axis=-1)
        sel1 = e_ids == i1[:, None]
        masked = jnp.where(sel1, -jnp.inf, probs)
        i2 = jnp.argmax(masked, axis=-1)
        p2 = jnp.max(masked, axis=-1)
        sel2 = e_ids == i2[:, None]
        wt = p1[:, None] * sel1.astype(jnp.float32) + p2[:, None] * sel2.astype(
            jnp.float32
        )
        wt_ref[...] = wt
        o_ref[...] = jnp.dot(wt, b2_ref[...], preferred_element_type=jnp.float32)

    inv_sqrt2 = 0.7071067811865476
    h = jnp.dot(xb, w1_ref[0], preferred_element_type=jnp.float32)
    h = h + b1_ref[0]
    g = 0.5 * h * (1.0 + jax.lax.erf(h * inv_sqrt2))
    y = jnp.dot(g, w2_ref[0], preferred_element_type=jnp.float32)
    wt = wt_ref[...]
    e_ids = jax.lax.broadcasted_iota(jnp.int32, wt.shape, 1)
    wcol = jnp.sum(
        jnp.where(e_ids == e, wt, 0.0), axis=-1, keepdims=True
    )
    o_ref[...] += wcol * y


def kernel(x, gate_w, w1, b1, w2, b2):
    xf = x.reshape(B * N, D)
    b1 = b1.reshape(E, 1, D)
    out = pl.pallas_call(
        _moe_block,
        grid=(NT, E),
        in_specs=[
            pl.BlockSpec((TB, D), lambda t, e: (t, 0)),
            pl.BlockSpec((D, E), lambda t, e: (0, 0)),
            pl.BlockSpec((1, D, D), lambda t, e: (e, 0, 0)),
            pl.BlockSpec((1, 1, D), lambda t, e: (e, 0, 0)),
            pl.BlockSpec((1, D, D), lambda t, e: (e, 0, 0)),
            pl.BlockSpec((E, D), lambda t, e: (0, 0)),
        ],
        out_specs=pl.BlockSpec((TB, D), lambda t, e: (t, 0)),
        out_shape=jax.ShapeDtypeStruct((B * N, D), jnp.float32),
        scratch_shapes=[pltpu.VMEM((TB, E), jnp.float32)],
        compiler_params=pltpu.CompilerParams(
            dimension_semantics=("arbitrary", "arbitrary"),
        ),
    )(xf, gate_w, w1, b1, w2, b2)
    return out.reshape(B, N, D)


# speedup vs baseline: 1.1936x; 1.1936x over previous
import jax
import jax.numpy as jnp
from jax.experimental import pallas as pl
from jax.experimental.pallas import tpu as pltpu

B, N, D, E, K = 2, 2048, 768, 8, 2
TB = 1024


def _moe_block(x_ref, gw_ref, w1_ref, b1_ref, w2_ref, b2_ref, o_ref):
    xb = x_ref[...]
    logits = jnp.dot(xb, gw_ref[...], preferred_element_type=jnp.float32)
    probs = jax.nn.softmax(logits, axis=-1)
    e_ids = jax.lax.broadcasted_iota(jnp.int32, probs.shape, 1)
    i1 = jnp.argmax(probs, axis=-1)
    p1 = jnp.max(probs, axis=-1)
    sel1 = e_ids == i1[:, None]
    masked = jnp.where(sel1, -jnp.inf, probs)
    i2 = jnp.argmax(masked, axis=-1)
    p2 = jnp.max(masked, axis=-1)
    sel2 = e_ids == i2[:, None]
    wt = p1[:, None] * sel1.astype(jnp.float32) + p2[:, None] * sel2.astype(
        jnp.float32
    )

    acc = jnp.zeros((xb.shape[0], D), jnp.float32)
    inv_sqrt2 = 0.7071067811865476
    for e in range(E):
        h = jnp.dot(xb, w1_ref[e], preferred_element_type=jnp.float32)
        h = h + b1_ref[e][None, :]
        h = 0.5 * h * (1.0 + jax.lax.erf(h * inv_sqrt2))
        y = jnp.dot(h, w2_ref[e], preferred_element_type=jnp.float32)
        y = y + b2_ref[e][None, :]
        acc = acc + wt[:, e][:, None] * y
    o_ref[...] = acc


def kernel(x, gate_w, w1, b1, w2, b2):
    xf = x.reshape(B * N, D)
    grid = (B * N // TB,)
    out = pl.pallas_call(
        _moe_block,
        grid=grid,
        in_specs=[
            pl.BlockSpec((TB, D), lambda i: (i, 0)),
            pl.BlockSpec((D, E), lambda i: (0, 0)),
            pl.BlockSpec((E, D, D), lambda i: (0, 0, 0)),
            pl.BlockSpec((E, D), lambda i: (0, 0)),
            pl.BlockSpec((E, D, D), lambda i: (0, 0, 0)),
            pl.BlockSpec((E, D), lambda i: (0, 0)),
        ],
        out_specs=pl.BlockSpec((TB, D), lambda i: (i, 0)),
        out_shape=jax.ShapeDtypeStruct((B * N, D), jnp.float32),
        compiler_params=pltpu.CompilerParams(
            dimension_semantics=("arbitrary",),
        ),
    )(xf, gate_w, w1, b1, w2, b2)
    return out.reshape(B, N, D)
